# baseline (speedup 1.0000x reference)
import functools

import jax
import jax.numpy as jnp
from jax import lax
from jax.experimental import pallas as pl
from jax.experimental.pallas import tpu as pltpu
from jax.experimental.pallas import tpu_sc as plsc

NC = 2
NS = 16
NW = NC * NS
CHUNK = 128
DEGW = 16


def _mesh():
    return plsc.VectorSubcoreMesh(core_axis_name="c", subcore_axis_name="s",
                                  num_cores=NC, num_subcores=NS)


def _make_deg(npad, epad):
    ept = epad // NW
    nchunks = ept // CHUNK
    spt = npad // NS

    @functools.partial(
        pl.kernel,
        out_type=jax.ShapeDtypeStruct((NC * npad,), jnp.float32),
        mesh=_mesh(),
        compiler_params=pltpu.CompilerParams(needs_layout_passes=False),
        scratch_types=[
            pltpu.VMEM_SHARED((NS * npad,), jnp.float32),
            pltpu.VMEM((npad,), jnp.float32),
            pltpu.VMEM((npad,), jnp.float32),
            pltpu.VMEM((spt,), jnp.float32),
            pltpu.VMEM((CHUNK,), jnp.int32),
            pltpu.VMEM((CHUNK,), jnp.int32),
            pltpu.SemaphoreType.DMA,
            pltpu.SemaphoreType.DMA,
        ],
    )
    def deg_kernel(dst_hbm, zeros_hbm, out_hbm, shared, hist, hbuf, rbuf,
                   dd0, dd1, is0, is1):
        D = [dd0, dd1]
        IS = [is0, is1]
        c = lax.axis_index("c")
        s = lax.axis_index("s")
        wid = c * NS + s
        pltpu.sync_copy(zeros_hbm, hist)
        ebase = wid * ept
        last = nchunks - 1
        ones = jnp.ones((16,), jnp.float32)

        def fetch(ci, q):
            pltpu.async_copy(dst_hbm.at[pl.ds(ebase + ci * CHUNK, CHUNK)],
                             D[q], IS[q])

        def wait_idx(q):
            pltpu.make_async_copy(dst_hbm.at[pl.ds(0, CHUNK)], D[q],
                                  IS[q]).wait()

        def dround(i, q):
            wait_idx(q)
            for j in range(CHUNK // 16):
                idx = D[q][pl.ds(j * 16, 16)]
                plsc.addupdate_scatter(hist, [idx], ones)
            fetch(jnp.minimum(i + 2, last), q)

        fetch(jnp.int32(0), 0)
        fetch(jnp.int32(1), 1)

        def body(g, carry):
            base = g * 2
            dround(base, 0)
            dround(base + 1, 1)
            return carry

        lax.fori_loop(0, nchunks // 2, body, 0)
        wait_idx(0)
        wait_idx(1)
        pltpu.sync_copy(hist, shared.at[pl.ds(s * npad, npad)])
        plsc.subcore_barrier()
        for t in range(NS):
            pltpu.sync_copy(shared.at[pl.ds(t * npad + s * spt, spt)],
                            hbuf.at[pl.ds(t * spt, spt)])

        def rbody(k, carry):
            v = hbuf[pl.ds(k * 16, 16)]
            for t in range(1, NS):
                v = v + hbuf[pl.ds(t * spt + k * 16, 16)]
            rbuf[pl.ds(k * 16, 16)] = v
            return carry

        lax.fori_loop(0, spt // 16, rbody, 0)
        pltpu.sync_copy(rbuf, out_hbm.at[pl.ds(c * npad + s * spt, spt)])

    return deg_kernel


def _make_agg(npad, epad, c0_eighths=4):
    ncht = epad // CHUNK
    nch0 = ncht * c0_eighths // (8 * NS)
    nch1 = ncht * (8 - c0_eighths) // (8 * NS)
    assert nch0 % 4 == 0 and nch1 % 4 == 0 and nch0 >= 8 and nch1 >= 8
    rpt = npad // NS

    @functools.partial(
        pl.kernel,
        out_type=jax.ShapeDtypeStruct((NC * npad, 128), jnp.float32),
        mesh=_mesh(),
        scratch_types=(
            [pltpu.VMEM_SHARED((npad, 128), jnp.float32)]
            + [pltpu.VMEM((CHUNK,), jnp.int32) for _ in range(8)]
            + [pltpu.VMEM((CHUNK, 128), jnp.float32) for _ in range(2)]
            + [pltpu.SemaphoreType.DMA for _ in range(8)]
        ),
    )
    def agg_kernel(g_hbm, src_hbm, dst_hbm, zeros_hbm, out_hbm,
                   acc, s0, s1, s2, s3, d0, d1, d2, d3, r0, r1,
                   i0, i1, i2, i3, g0, g1, ss0, ss1):
        S = [s0, s1, s2, s3]
        D = [d0, d1, d2, d3]
        R = [r0, r1]
        IS = [i0, i1, i2, i3]
        GS = [g0, g1]
        SS = [ss0, ss1]
        c = lax.axis_index("c")
        s = lax.axis_index("s")
        pltpu.sync_copy(zeros_hbm, acc.at[pl.ds(s * rpt, rpt)])
        plsc.subcore_barrier()
        nch_c = jnp.where(c == 0, nch0, nch1)
        cbase = c * NS * nch0 + s * nch_c
        ebase = cbase * CHUNK
        last = nch_c - 1

        def fetch(ci, q):
            b = ebase + ci * CHUNK
            pltpu.async_copy(src_hbm.at[pl.ds(b, CHUNK)], S[q], IS[q])
            pltpu.async_copy(dst_hbm.at[pl.ds(b, CHUNK)], D[q], IS[q])

        def wait_idx(q):
            pltpu.make_async_copy(src_hbm.at[pl.ds(0, CHUNK)], S[q],
                                  IS[q]).wait()
            pltpu.make_async_copy(dst_hbm.at[pl.ds(0, CHUNK)], D[q],
                                  IS[q]).wait()

        def wait_scat(b):
            pltpu.make_async_copy(g_hbm.at[pl.ds(0, CHUNK)], R[b],
                                  SS[b]).wait()

        def round_(i, j, first_pair=False):
            b = j % 2
            if not first_pair:
                wait_scat(b)
            fetch(jnp.minimum(i + 2, last), (j + 2) % 4)
            wait_idx(j)
            pltpu.async_copy(g_hbm.at[S[j]], R[b], GS[b]).wait()
            pltpu.async_copy(R[b], acc.at[D[j]], SS[b], add=True)

        fetch(jnp.int32(0), 0)
        fetch(jnp.int32(1), 1)
        round_(jnp.int32(0), 0, first_pair=True)
        round_(jnp.int32(1), 1, first_pair=True)
        round_(jnp.int32(2), 2)
        round_(jnp.int32(3), 3)

        def gbody(g, carry):
            base = g * 4
            round_(base + 0, 0)
            round_(base + 1, 1)
            round_(base + 2, 2)
            round_(base + 3, 3)
            return carry

        lax.fori_loop(1, nch_c // 4, gbody, 0)
        wait_scat(0)
        wait_scat(1)
        wait_idx(0)
        wait_idx(1)
        plsc.subcore_barrier()
        pltpu.sync_copy(acc.at[pl.ds(s * rpt, rpt)],
                        out_hbm.at[pl.ds(c * npad + s * rpt, rpt)])

    return agg_kernel


def _tc1(x_p, W1, deg_col, bn):
    npad = x_p.shape[0]
    nb = npad // bn

    def body(x_ref, w_ref, d_ref, g_ref):
        dinv = lax.rsqrt(d_ref[...])
        g_ref[...] = jnp.dot(x_ref[...], w_ref[...],
                             preferred_element_type=jnp.float32) * dinv

    return pl.pallas_call(
        body,
        grid=(nb,),
        in_specs=[
            pl.BlockSpec((bn, 128), lambda i: (i, 0)),
            pl.BlockSpec((128, 128), lambda i: (0, 0)),
            pl.BlockSpec((bn, 1), lambda i: (i, 0)),
        ],
        out_specs=pl.BlockSpec((bn, 128), lambda i: (i, 0)),
        out_shape=jax.ShapeDtypeStruct((npad, 128), jnp.float32),
    )(x_p, W1, deg_col)


def _tc2(parts, g1, deg_col, b1, W2, bn):
    npad = g1.shape[0]
    nb = npad // bn

    def body(p0, p1, g_ref, d_ref, b_ref, w_ref, out_ref):
        dinv = lax.rsqrt(d_ref[...])
        h = jnp.maximum((p0[...] + p1[...] + g_ref[...]) * dinv + b_ref[...],
                        0.0)
        out_ref[...] = jnp.dot(h, w_ref[...],
                               preferred_element_type=jnp.float32) * dinv

    return pl.pallas_call(
        body,
        grid=(nb,),
        in_specs=[
            pl.BlockSpec((bn, 128), lambda i: (i, 0)),
            pl.BlockSpec((bn, 128), lambda i, _nb=nb: (i + _nb, 0)),
            pl.BlockSpec((bn, 128), lambda i: (i, 0)),
            pl.BlockSpec((bn, 1), lambda i: (i, 0)),
            pl.BlockSpec((1, 128), lambda i: (0, 0)),
            pl.BlockSpec((128, 128), lambda i: (0, 0)),
        ],
        out_specs=pl.BlockSpec((bn, 128), lambda i: (i, 0)),
        out_shape=jax.ShapeDtypeStruct((npad, 128), jnp.float32),
    )(parts, parts, g1, deg_col, b1, W2)


def _tc3(parts, g2, deg_col, b2, Wl, bl, bn):
    npad = g2.shape[0]
    nb = npad // bn
    cdim = Wl.shape[1]

    def body(p0, p1, g_ref, d_ref, b_ref, w_ref, bl_ref, out_ref):
        dinv = lax.rsqrt(d_ref[...])
        h = jnp.maximum((p0[...] + p1[...] + g_ref[...]) * dinv + b_ref[...],
                        0.0)
        out_ref[...] = jnp.dot(h, w_ref[...],
                               preferred_element_type=jnp.float32) + bl_ref[...]

    return pl.pallas_call(
        body,
        grid=(nb,),
        in_specs=[
            pl.BlockSpec((bn, 128), lambda i: (i, 0)),
            pl.BlockSpec((bn, 128), lambda i, _nb=nb: (i + _nb, 0)),
            pl.BlockSpec((bn, 128), lambda i: (i, 0)),
            pl.BlockSpec((bn, 1), lambda i: (i, 0)),
            pl.BlockSpec((1, 128), lambda i: (0, 0)),
            pl.BlockSpec((128, cdim), lambda i: (0, 0)),
            pl.BlockSpec((1, cdim), lambda i: (0, 0)),
        ],
        out_specs=pl.BlockSpec((bn, cdim), lambda i: (i, 0)),
        out_shape=jax.ShapeDtypeStruct((npad, cdim), jnp.float32),
    )(parts, parts, g2, deg_col, b2, Wl, bl)


def kernel(x, edge_index, W1, b1, W2, b2, Wl, bl):
    n = x.shape[0]
    e = edge_index.shape[1]
    npad = ((n + 1 + 511) // 512) * 512
    epad = -(-e // (NW * CHUNK * 4)) * (NW * CHUNK * 4)
    rpt = npad // NS
    bn = 1024 if npad % 1024 == 0 else 512

    src = edge_index[0].astype(jnp.int32)
    dst = edge_index[1].astype(jnp.int32)
    src_p = jnp.concatenate([src, jnp.zeros((epad - e,), jnp.int32)])
    dst_p = jnp.concatenate([dst, jnp.full((epad - e,), n, jnp.int32)])
    x_p = jnp.pad(x, ((0, npad - n), (0, 0)))

    zeros128 = jnp.zeros((rpt, 128), jnp.float32)
    zeros1d = jnp.zeros((npad,), jnp.float32)

    deg_call = _make_deg(npad, epad)
    agg_call = _make_agg(npad, epad, c0_eighths=2)

    dparts = deg_call(dst_p, zeros1d)
    deg_col = (dparts[:npad] + dparts[npad:] + 1.0).reshape(npad, 1)

    g1 = _tc1(x_p, W1, deg_col, bn)
    p1 = agg_call(g1, src_p, dst_p, zeros128)
    g2 = _tc2(p1, g1, deg_col, b1.reshape(1, -1), W2, bn)
    p2 = agg_call(g2, src_p, dst_p, zeros128)
    out = _tc3(p2, g2, deg_col, b2.reshape(1, -1), Wl, bl.reshape(1, -1), bn)
    return out[:n]


# speedup vs baseline: 1.1432x; 1.1432x over previous
import functools

import jax
import jax.numpy as jnp
from jax import lax
from jax.experimental import pallas as pl
from jax.experimental.pallas import tpu as pltpu
from jax.experimental.pallas import tpu_sc as plsc

NC = 2
NS = 16
NW = NC * NS
CHUNK = 128
DEGW = 16


def _mesh():
    return plsc.VectorSubcoreMesh(core_axis_name="c", subcore_axis_name="s",
                                  num_cores=NC, num_subcores=NS)


def _make_deg(npad, epad):
    ept = epad // NW
    nchunks = ept // CHUNK
    spt = npad // NS

    @functools.partial(
        pl.kernel,
        out_type=jax.ShapeDtypeStruct((NC * npad,), jnp.float32),
        mesh=_mesh(),
        compiler_params=pltpu.CompilerParams(needs_layout_passes=False),
        scratch_types=[
            pltpu.VMEM_SHARED((NS * npad,), jnp.float32),
            pltpu.VMEM((npad,), jnp.float32),
            pltpu.VMEM((npad,), jnp.float32),
            pltpu.VMEM((spt,), jnp.float32),
            pltpu.VMEM((CHUNK,), jnp.int32),
            pltpu.VMEM((CHUNK,), jnp.int32),
            pltpu.SemaphoreType.DMA,
            pltpu.SemaphoreType.DMA,
        ],
    )
    def deg_kernel(dst_hbm, zeros_hbm, out_hbm, shared, hist, hbuf, rbuf,
                   dd0, dd1, is0, is1):
        D = [dd0, dd1]
        IS = [is0, is1]
        c = lax.axis_index("c")
        s = lax.axis_index("s")
        wid = c * NS + s
        pltpu.sync_copy(zeros_hbm, hist)
        ebase = wid * ept
        last = nchunks - 1
        ones = jnp.ones((16,), jnp.float32)

        def fetch(ci, q):
            pltpu.async_copy(dst_hbm.at[pl.ds(ebase + ci * CHUNK, CHUNK)],
                             D[q], IS[q])

        def wait_idx(q):
            pltpu.make_async_copy(dst_hbm.at[pl.ds(0, CHUNK)], D[q],
                                  IS[q]).wait()

        def dround(i, q):
            wait_idx(q)
            for j in range(CHUNK // 16):
                idx = D[q][pl.ds(j * 16, 16)]
                plsc.addupdate_scatter(hist, [idx], ones)
            fetch(jnp.minimum(i + 2, last), q)

        fetch(jnp.int32(0), 0)
        fetch(jnp.int32(1), 1)

        def body(g, carry):
            base = g * 2
            dround(base, 0)
            dround(base + 1, 1)
            return carry

        lax.fori_loop(0, nchunks // 2, body, 0)
        wait_idx(0)
        wait_idx(1)
        pltpu.sync_copy(hist, shared.at[pl.ds(s * npad, npad)])
        plsc.subcore_barrier()
        for t in range(NS):
            pltpu.sync_copy(shared.at[pl.ds(t * npad + s * spt, spt)],
                            hbuf.at[pl.ds(t * spt, spt)])

        def rbody(k, carry):
            v = hbuf[pl.ds(k * 16, 16)]
            for t in range(1, NS):
                v = v + hbuf[pl.ds(t * spt + k * 16, 16)]
            rbuf[pl.ds(k * 16, 16)] = v
            return carry

        lax.fori_loop(0, spt // 16, rbody, 0)
        pltpu.sync_copy(rbuf, out_hbm.at[pl.ds(c * npad + s * spt, spt)])

    return deg_kernel


def _make_agg(npad, epad, c0_eighths=4):
    ncht = epad // CHUNK
    nch0 = ncht * c0_eighths // (8 * NS)
    nch1 = ncht * (8 - c0_eighths) // (8 * NS)
    assert nch0 % 4 == 0 and nch1 % 4 == 0 and nch0 >= 8 and nch1 >= 8
    rpt = npad // NS

    @functools.partial(
        pl.kernel,
        out_type=jax.ShapeDtypeStruct((NC * npad, 128), jnp.float32),
        mesh=_mesh(),
        scratch_types=(
            [pltpu.VMEM_SHARED((npad, 128), jnp.float32)]
            + [pltpu.VMEM((CHUNK,), jnp.int32) for _ in range(8)]
            + [pltpu.VMEM((CHUNK, 128), jnp.float32) for _ in range(2)]
            + [pltpu.SemaphoreType.DMA for _ in range(8)]
        ),
    )
    def agg_kernel(g_hbm, src_hbm, dst_hbm, zeros_hbm, out_hbm,
                   acc, s0, s1, s2, s3, d0, d1, d2, d3, r0, r1,
                   i0, i1, i2, i3, g0, g1, ss0, ss1):
        S = [s0, s1, s2, s3]
        D = [d0, d1, d2, d3]
        R = [r0, r1]
        IS = [i0, i1, i2, i3]
        GS = [g0, g1]
        SS = [ss0, ss1]
        c = lax.axis_index("c")
        s = lax.axis_index("s")
        pltpu.sync_copy(zeros_hbm, acc.at[pl.ds(s * rpt, rpt)])
        plsc.subcore_barrier()
        nch_c = jnp.where(c == 0, nch0, nch1)
        cbase = c * NS * nch0 + s * nch_c
        ebase = cbase * CHUNK
        last = nch_c - 1

        def fetch(ci, q):
            b = ebase + ci * CHUNK
            pltpu.async_copy(src_hbm.at[pl.ds(b, CHUNK)], S[q], IS[q])
            pltpu.async_copy(dst_hbm.at[pl.ds(b, CHUNK)], D[q], IS[q])

        def wait_idx(q):
            pltpu.make_async_copy(src_hbm.at[pl.ds(0, CHUNK)], S[q],
                                  IS[q]).wait()
            pltpu.make_async_copy(dst_hbm.at[pl.ds(0, CHUNK)], D[q],
                                  IS[q]).wait()

        def wait_scat(b):
            pltpu.make_async_copy(g_hbm.at[pl.ds(0, CHUNK)], R[b],
                                  SS[b]).wait()

        def round_(i, j, first_pair=False):
            b = j % 2
            if not first_pair:
                wait_scat(b)
            fetch(jnp.minimum(i + 2, last), (j + 2) % 4)
            wait_idx(j)
            pltpu.async_copy(g_hbm.at[S[j]], R[b], GS[b]).wait()
            pltpu.async_copy(R[b], acc.at[D[j]], SS[b], add=True)

        fetch(jnp.int32(0), 0)
        fetch(jnp.int32(1), 1)
        round_(jnp.int32(0), 0, first_pair=True)
        round_(jnp.int32(1), 1, first_pair=True)
        round_(jnp.int32(2), 2)
        round_(jnp.int32(3), 3)

        def gbody(g, carry):
            base = g * 4
            round_(base + 0, 0)
            round_(base + 1, 1)
            round_(base + 2, 2)
            round_(base + 3, 3)
            return carry

        lax.fori_loop(1, nch_c // 4, gbody, 0)
        wait_scat(0)
        wait_scat(1)
        wait_idx(0)
        wait_idx(1)
        plsc.subcore_barrier()
        pltpu.sync_copy(acc.at[pl.ds(s * rpt, rpt)],
                        out_hbm.at[pl.ds(c * npad + s * rpt, rpt)])

    return agg_kernel


def _tc1(x_p, W1, deg_col, bn):
    npad = x_p.shape[0]
    nb = npad // bn

    def body(x_ref, w_ref, d_ref, g_ref):
        dinv = lax.rsqrt(d_ref[...])
        g_ref[...] = jnp.dot(x_ref[...], w_ref[...],
                             preferred_element_type=jnp.float32) * dinv

    return pl.pallas_call(
        body,
        grid=(nb,),
        in_specs=[
            pl.BlockSpec((bn, 128), lambda i: (i, 0)),
            pl.BlockSpec((128, 128), lambda i: (0, 0)),
            pl.BlockSpec((bn, 1), lambda i: (i, 0)),
        ],
        out_specs=pl.BlockSpec((bn, 128), lambda i: (i, 0)),
        out_shape=jax.ShapeDtypeStruct((npad, 128), jnp.float32),
    )(x_p, W1, deg_col)


def _tc2(parts, g1, deg_col, b1, W2, bn):
    npad = g1.shape[0]
    nb = npad // bn

    def body(p0, p1, g_ref, d_ref, b_ref, w_ref, out_ref):
        dinv = lax.rsqrt(d_ref[...])
        h = jnp.maximum((p0[...] + p1[...] + g_ref[...]) * dinv + b_ref[...],
                        0.0)
        out_ref[...] = jnp.dot(h, w_ref[...],
                               preferred_element_type=jnp.float32) * dinv

    return pl.pallas_call(
        body,
        grid=(nb,),
        in_specs=[
            pl.BlockSpec((bn, 128), lambda i: (i, 0)),
            pl.BlockSpec((bn, 128), lambda i, _nb=nb: (i + _nb, 0)),
            pl.BlockSpec((bn, 128), lambda i: (i, 0)),
            pl.BlockSpec((bn, 1), lambda i: (i, 0)),
            pl.BlockSpec((1, 128), lambda i: (0, 0)),
            pl.BlockSpec((128, 128), lambda i: (0, 0)),
        ],
        out_specs=pl.BlockSpec((bn, 128), lambda i: (i, 0)),
        out_shape=jax.ShapeDtypeStruct((npad, 128), jnp.float32),
    )(parts, parts, g1, deg_col, b1, W2)


def _tc3(parts, g2, deg_col, b2, Wl, bl, bn):
    npad = g2.shape[0]
    nb = npad // bn
    cdim = Wl.shape[1]

    def body(p0, p1, g_ref, d_ref, b_ref, w_ref, bl_ref, out_ref):
        dinv = lax.rsqrt(d_ref[...])
        h = jnp.maximum((p0[...] + p1[...] + g_ref[...]) * dinv + b_ref[...],
                        0.0)
        out_ref[...] = jnp.dot(h, w_ref[...],
                               preferred_element_type=jnp.float32) + bl_ref[...]

    return pl.pallas_call(
        body,
        grid=(nb,),
        in_specs=[
            pl.BlockSpec((bn, 128), lambda i: (i, 0)),
            pl.BlockSpec((bn, 128), lambda i, _nb=nb: (i + _nb, 0)),
            pl.BlockSpec((bn, 128), lambda i: (i, 0)),
            pl.BlockSpec((bn, 1), lambda i: (i, 0)),
            pl.BlockSpec((1, 128), lambda i: (0, 0)),
            pl.BlockSpec((128, cdim), lambda i: (0, 0)),
            pl.BlockSpec((1, cdim), lambda i: (0, 0)),
        ],
        out_specs=pl.BlockSpec((bn, cdim), lambda i: (i, 0)),
        out_shape=jax.ShapeDtypeStruct((npad, cdim), jnp.float32),
    )(parts, parts, g2, deg_col, b2, Wl, bl)


def kernel(x, edge_index, W1, b1, W2, b2, Wl, bl):
    n = x.shape[0]
    e = edge_index.shape[1]
    npad = ((n + 1 + 511) // 512) * 512
    epad = -(-e // (NW * CHUNK * 4)) * (NW * CHUNK * 4)
    rpt = npad // NS
    bn = 1024 if npad % 1024 == 0 else 512

    src = edge_index[0].astype(jnp.int32)
    dst = edge_index[1].astype(jnp.int32)
    src_p = jnp.concatenate([src, jnp.zeros((epad - e,), jnp.int32)])
    dst_p = jnp.concatenate([dst, jnp.full((epad - e,), n, jnp.int32)])
    x_p = jnp.pad(x, ((0, npad - n), (0, 0)))

    zeros128 = jnp.zeros((rpt, 128), jnp.float32)
    zeros1d = jnp.zeros((npad,), jnp.float32)

    deg_call = _make_deg(npad, epad)
    agg_call = _make_agg(npad, epad, c0_eighths=6)

    dparts = deg_call(dst_p, zeros1d)
    deg_col = (dparts[:npad] + dparts[npad:] + 1.0).reshape(npad, 1)

    g1 = _tc1(x_p, W1, deg_col, bn)
    p1 = agg_call(g1, src_p, dst_p, zeros128)
    g2 = _tc2(p1, g1, deg_col, b1.reshape(1, -1), W2, bn)
    p2 = agg_call(g2, src_p, dst_p, zeros128)
    out = _tc3(p2, g2, deg_col, b2.reshape(1, -1), Wl, bl.reshape(1, -1), bn)
    return out[:n]


# speedup vs baseline: 1.1764x; 1.0291x over previous
import functools

import jax
import jax.numpy as jnp
from jax import lax
from jax.experimental import pallas as pl
from jax.experimental.pallas import tpu as pltpu
from jax.experimental.pallas import tpu_sc as plsc

NC = 2
NS = 16
NW = NC * NS
CHUNK = 128
DEGW = 16


def _mesh():
    return plsc.VectorSubcoreMesh(core_axis_name="c", subcore_axis_name="s",
                                  num_cores=NC, num_subcores=NS)


def _make_deg(npad, epad):
    ept = epad // NW
    nchunks = ept // CHUNK
    spt = npad // NS

    @functools.partial(
        pl.kernel,
        out_type=jax.ShapeDtypeStruct((NC * npad,), jnp.float32),
        mesh=_mesh(),
        compiler_params=pltpu.CompilerParams(needs_layout_passes=False),
        scratch_types=[
            pltpu.VMEM_SHARED((NS * npad,), jnp.float32),
            pltpu.VMEM((npad,), jnp.float32),
            pltpu.VMEM((npad,), jnp.float32),
            pltpu.VMEM((spt,), jnp.float32),
            pltpu.VMEM((CHUNK,), jnp.int32),
            pltpu.VMEM((CHUNK,), jnp.int32),
            pltpu.SemaphoreType.DMA,
            pltpu.SemaphoreType.DMA,
        ],
    )
    def deg_kernel(dst_hbm, zeros_hbm, out_hbm, shared, hist, hbuf, rbuf,
                   dd0, dd1, is0, is1):
        D = [dd0, dd1]
        IS = [is0, is1]
        c = lax.axis_index("c")
        s = lax.axis_index("s")
        wid = c * NS + s
        pltpu.sync_copy(zeros_hbm, hist)
        ebase = wid * ept
        last = nchunks - 1
        ones = jnp.ones((16,), jnp.float32)

        def fetch(ci, q):
            pltpu.async_copy(dst_hbm.at[pl.ds(ebase + ci * CHUNK, CHUNK)],
                             D[q], IS[q])

        def wait_idx(q):
            pltpu.make_async_copy(dst_hbm.at[pl.ds(0, CHUNK)], D[q],
                                  IS[q]).wait()

        def dround(i, q):
            wait_idx(q)
            for j in range(CHUNK // 16):
                idx = D[q][pl.ds(j * 16, 16)]
                plsc.addupdate_scatter(hist, [idx], ones)
            fetch(jnp.minimum(i + 2, last), q)

        fetch(jnp.int32(0), 0)
        fetch(jnp.int32(1), 1)

        def body(g, carry):
            base = g * 2
            dround(base, 0)
            dround(base + 1, 1)
            return carry

        lax.fori_loop(0, nchunks // 2, body, 0)
        wait_idx(0)
        wait_idx(1)
        pltpu.sync_copy(hist, shared.at[pl.ds(s * npad, npad)])
        plsc.subcore_barrier()
        for t in range(NS):
            pltpu.sync_copy(shared.at[pl.ds(t * npad + s * spt, spt)],
                            hbuf.at[pl.ds(t * spt, spt)])

        def rbody(k, carry):
            v = hbuf[pl.ds(k * 16, 16)]
            for t in range(1, NS):
                v = v + hbuf[pl.ds(t * spt + k * 16, 16)]
            rbuf[pl.ds(k * 16, 16)] = v
            return carry

        lax.fori_loop(0, spt // 16, rbody, 0)
        pltpu.sync_copy(rbuf, out_hbm.at[pl.ds(c * npad + s * spt, spt)])

    return deg_kernel


def _make_agg(npad, epad, c0_frac=0.5):
    ncht = epad // CHUNK
    nch0 = int(round(ncht * c0_frac / (NS * 4))) * 4
    nch1 = ncht // NS - nch0
    assert nch0 % 4 == 0 and nch1 % 4 == 0 and nch0 >= 8 and nch1 >= 8
    rpt = npad // NS

    @functools.partial(
        pl.kernel,
        out_type=jax.ShapeDtypeStruct((NC * npad, 128), jnp.float32),
        mesh=_mesh(),
        scratch_types=(
            [pltpu.VMEM_SHARED((npad, 128), jnp.float32)]
            + [pltpu.VMEM((CHUNK,), jnp.int32) for _ in range(8)]
            + [pltpu.VMEM((CHUNK, 128), jnp.float32) for _ in range(2)]
            + [pltpu.SemaphoreType.DMA for _ in range(8)]
        ),
    )
    def agg_kernel(g_hbm, src_hbm, dst_hbm, zeros_hbm, out_hbm,
                   acc, s0, s1, s2, s3, d0, d1, d2, d3, r0, r1,
                   i0, i1, i2, i3, g0, g1, ss0, ss1):
        S = [s0, s1, s2, s3]
        D = [d0, d1, d2, d3]
        R = [r0, r1]
        IS = [i0, i1, i2, i3]
        GS = [g0, g1]
        SS = [ss0, ss1]
        c = lax.axis_index("c")
        s = lax.axis_index("s")
        pltpu.sync_copy(zeros_hbm, acc.at[pl.ds(s * rpt, rpt)])
        plsc.subcore_barrier()
        nch_c = jnp.where(c == 0, nch0, nch1)
        cbase = c * NS * nch0 + s * nch_c
        ebase = cbase * CHUNK
        last = nch_c - 1

        def fetch(ci, q):
            b = ebase + ci * CHUNK
            pltpu.async_copy(src_hbm.at[pl.ds(b, CHUNK)], S[q], IS[q])
            pltpu.async_copy(dst_hbm.at[pl.ds(b, CHUNK)], D[q], IS[q])

        def wait_idx(q):
            pltpu.make_async_copy(src_hbm.at[pl.ds(0, CHUNK)], S[q],
                                  IS[q]).wait()
            pltpu.make_async_copy(dst_hbm.at[pl.ds(0, CHUNK)], D[q],
                                  IS[q]).wait()

        def wait_scat(b):
            pltpu.make_async_copy(g_hbm.at[pl.ds(0, CHUNK)], R[b],
                                  SS[b]).wait()

        def round_(i, j, first_pair=False):
            b = j % 2
            if not first_pair:
                wait_scat(b)
            fetch(jnp.minimum(i + 2, last), (j + 2) % 4)
            wait_idx(j)
            pltpu.async_copy(g_hbm.at[S[j]], R[b], GS[b]).wait()
            pltpu.async_copy(R[b], acc.at[D[j]], SS[b], add=True)

        fetch(jnp.int32(0), 0)
        fetch(jnp.int32(1), 1)
        round_(jnp.int32(0), 0, first_pair=True)
        round_(jnp.int32(1), 1, first_pair=True)
        round_(jnp.int32(2), 2)
        round_(jnp.int32(3), 3)

        def gbody(g, carry):
            base = g * 4
            round_(base + 0, 0)
            round_(base + 1, 1)
            round_(base + 2, 2)
            round_(base + 3, 3)
            return carry

        lax.fori_loop(1, nch_c // 4, gbody, 0)
        wait_scat(0)
        wait_scat(1)
        wait_idx(0)
        wait_idx(1)
        plsc.subcore_barrier()
        pltpu.sync_copy(acc.at[pl.ds(s * rpt, rpt)],
                        out_hbm.at[pl.ds(c * npad + s * rpt, rpt)])

    return agg_kernel


def _tc1(x_p, W1, deg_col, bn):
    npad = x_p.shape[0]
    nb = npad // bn

    def body(x_ref, w_ref, d_ref, g_ref):
        dinv = lax.rsqrt(d_ref[...])
        g_ref[...] = jnp.dot(x_ref[...], w_ref[...],
                             preferred_element_type=jnp.float32) * dinv

    return pl.pallas_call(
        body,
        grid=(nb,),
        in_specs=[
            pl.BlockSpec((bn, 128), lambda i: (i, 0)),
            pl.BlockSpec((128, 128), lambda i: (0, 0)),
            pl.BlockSpec((bn, 1), lambda i: (i, 0)),
        ],
        out_specs=pl.BlockSpec((bn, 128), lambda i: (i, 0)),
        out_shape=jax.ShapeDtypeStruct((npad, 128), jnp.float32),
    )(x_p, W1, deg_col)


def _tc2(parts, g1, deg_col, b1, W2, bn):
    npad = g1.shape[0]
    nb = npad // bn

    def body(p0, p1, g_ref, d_ref, b_ref, w_ref, out_ref):
        dinv = lax.rsqrt(d_ref[...])
        h = jnp.maximum((p0[...] + p1[...] + g_ref[...]) * dinv + b_ref[...],
                        0.0)
        out_ref[...] = jnp.dot(h, w_ref[...],
                               preferred_element_type=jnp.float32) * dinv

    return pl.pallas_call(
        body,
        grid=(nb,),
        in_specs=[
            pl.BlockSpec((bn, 128), lambda i: (i, 0)),
            pl.BlockSpec((bn, 128), lambda i, _nb=nb: (i + _nb, 0)),
            pl.BlockSpec((bn, 128), lambda i: (i, 0)),
            pl.BlockSpec((bn, 1), lambda i: (i, 0)),
            pl.BlockSpec((1, 128), lambda i: (0, 0)),
            pl.BlockSpec((128, 128), lambda i: (0, 0)),
        ],
        out_specs=pl.BlockSpec((bn, 128), lambda i: (i, 0)),
        out_shape=jax.ShapeDtypeStruct((npad, 128), jnp.float32),
    )(parts, parts, g1, deg_col, b1, W2)


def _tc3(parts, g2, deg_col, b2, Wl, bl, bn):
    npad = g2.shape[0]
    nb = npad // bn
    cdim = Wl.shape[1]

    def body(p0, p1, g_ref, d_ref, b_ref, w_ref, bl_ref, out_ref):
        dinv = lax.rsqrt(d_ref[...])
        h = jnp.maximum((p0[...] + p1[...] + g_ref[...]) * dinv + b_ref[...],
                        0.0)
        out_ref[...] = jnp.dot(h, w_ref[...],
                               preferred_element_type=jnp.float32) + bl_ref[...]

    return pl.pallas_call(
        body,
        grid=(nb,),
        in_specs=[
            pl.BlockSpec((bn, 128), lambda i: (i, 0)),
            pl.BlockSpec((bn, 128), lambda i, _nb=nb: (i + _nb, 0)),
            pl.BlockSpec((bn, 128), lambda i: (i, 0)),
            pl.BlockSpec((bn, 1), lambda i: (i, 0)),
            pl.BlockSpec((1, 128), lambda i: (0, 0)),
            pl.BlockSpec((128, cdim), lambda i: (0, 0)),
            pl.BlockSpec((1, cdim), lambda i: (0, 0)),
        ],
        out_specs=pl.BlockSpec((bn, cdim), lambda i: (i, 0)),
        out_shape=jax.ShapeDtypeStruct((npad, cdim), jnp.float32),
    )(parts, parts, g2, deg_col, b2, Wl, bl)


def kernel(x, edge_index, W1, b1, W2, b2, Wl, bl):
    n = x.shape[0]
    e = edge_index.shape[1]
    npad = ((n + 1 + 511) // 512) * 512
    epad = -(-e // (NW * CHUNK * 4)) * (NW * CHUNK * 4)
    rpt = npad // NS
    bn = 1024 if npad % 1024 == 0 else 512

    src = edge_index[0].astype(jnp.int32)
    dst = edge_index[1].astype(jnp.int32)
    src_p = jnp.concatenate([src, jnp.zeros((epad - e,), jnp.int32)])
    dst_p = jnp.concatenate([dst, jnp.full((epad - e,), n, jnp.int32)])
    x_p = jnp.pad(x, ((0, npad - n), (0, 0)))

    zeros128 = jnp.zeros((rpt, 128), jnp.float32)
    zeros1d = jnp.zeros((npad,), jnp.float32)

    deg_call = _make_deg(npad, epad)
    agg_call = _make_agg(npad, epad, c0_frac=0.95)

    dparts = deg_call(dst_p, zeros1d)
    deg_col = (dparts[:npad] + dparts[npad:] + 1.0).reshape(npad, 1)

    g1 = _tc1(x_p, W1, deg_col, bn)
    p1 = agg_call(g1, src_p, dst_p, zeros128)
    g2 = _tc2(p1, g1, deg_col, b1.reshape(1, -1), W2, bn)
    p2 = agg_call(g2, src_p, dst_p, zeros128)
    out = _tc3(p2, g2, deg_col, b2.reshape(1, -1), Wl, bl.reshape(1, -1), bn)
    return out[:n]
